# baseline (speedup 1.0000x reference)
import jax
import jax.numpy as jnp
from jax import lax
from jax.experimental import pallas as pl
from jax.experimental.pallas import tpu as pltpu
from jax.experimental.pallas import tpu_sc as plsc

N = 10000
E = 320000
D = 128
DH = 64
NC = 2
NS = 16
CHUNK = 40
KBUF = 16
NBLK = 32
NCHUNK = NBLK * KBUF
EPT = NCHUNK * CHUNK
E_PAD = EPT * NS
SH = 625
ROWS_PT = 640
N_PAD = ROWS_PT * NS
TRASH = N
ZROWS = 64


def _mm_body(x_ref, wt_ref, b_ref, o_ref):
    xb = x_ref[...]
    for c in range(NC):
        o_ref[c] = (
            jnp.dot(xb, wt_ref[c], preferred_element_type=jnp.float32)
            + b_ref[c][None, :]
        )


def _linear(x, wts, bs):
    bn = 400
    grid = N // bn
    return pl.pallas_call(
        _mm_body,
        grid=(grid,),
        in_specs=[
            pl.BlockSpec((bn, D), lambda i: (i, 0)),
            pl.BlockSpec((NC, D, DH), lambda i: (0, 0, 0)),
            pl.BlockSpec((NC, DH), lambda i: (0, 0)),
        ],
        out_specs=pl.BlockSpec((NC, bn, DH), lambda i: (0, i, 0)),
        out_shape=jax.ShapeDtypeStruct((NC, N, DH), jnp.float32),
    )(x, wts, bs)


def _sc_body(y2, srcr, dstr, out2, tabS, acc, idx_b, rows_v, zero_v,
             sem_i, sem_g, sem_s):
    c = lax.axis_index("c")
    s = lax.axis_index("s")

    def _zfill(r, carry):
        for t in range(DH // 16):
            zero_v[r, pl.ds(t * 16, 16)] = jnp.zeros((16,), jnp.float32)
        return carry

    lax.fori_loop(0, ZROWS, _zfill, 0)

    def clear(tab):
        for z in range(ROWS_PT // ZROWS):
            pltpu.sync_copy(
                zero_v, tab.at[pl.ds(s * ROWS_PT + z * ZROWS, ZROWS)])

    pltpu.sync_copy(y2.at[c, pl.ds(s * SH, SH)], tabS.at[pl.ds(s * SH, SH)])
    clear(acc)
    plsc.subcore_barrier()

    def one_round(tab, ac):
        pltpu.sync_copy(srcr.at[s, 0], idx_b.at[0, 0])
        pltpu.sync_copy(dstr.at[s, 0], idx_b.at[0, 1])

        def do_block(b, p):
            nb = jnp.minimum(b + 1, NBLK - 1)
            pi = pltpu.async_copy(srcr.at[s, nb], idx_b.at[1 - p, 0], sem_i)
            pd = pltpu.async_copy(dstr.at[s, nb], idx_b.at[1 - p, 1], sem_i)
            gathers = []
            for k in range(KBUF):
                gathers.append(pltpu.async_copy(
                    tab.at[idx_b.at[p, 0, k]], rows_v.at[k], sem_g))
            scatters = []
            for k in range(KBUF):
                gathers[k].wait()
                scatters.append(pltpu.async_copy(
                    rows_v.at[k], ac.at[idx_b.at[p, 1, k]], sem_s,
                    add=True))
            for k in range(KBUF):
                scatters[k].wait()
            pi.wait()
            pd.wait()

        def pair(bp, carry):
            do_block(bp * 2, 0)
            do_block(bp * 2 + 1, 1)
            return carry

        lax.fori_loop(0, NBLK // 2, pair, 0)
        plsc.subcore_barrier()

    one_round(tabS, acc)
    clear(tabS)
    plsc.subcore_barrier()
    one_round(acc, tabS)
    clear(acc)
    plsc.subcore_barrier()
    one_round(tabS, acc)

    pltpu.sync_copy(acc.at[pl.ds(s * SH, SH)],
                    out2.at[c, pl.ds(s * SH, SH)])


def _spmm3(y2, srcr, dstr):
    mesh = plsc.VectorSubcoreMesh(core_axis_name="c", subcore_axis_name="s")
    return pl.kernel(
        _sc_body,
        out_type=jax.ShapeDtypeStruct((NC, N, DH), jnp.float32),
        mesh=mesh,
        compiler_params=pltpu.CompilerParams(use_tc_tiling_on_sc=False),
        scratch_types=[
            pltpu.VMEM_SHARED((N_PAD, DH), jnp.float32),
            pltpu.VMEM_SHARED((N_PAD, DH), jnp.float32),
            pltpu.VMEM((2, 2, KBUF, CHUNK), jnp.int32),
            pltpu.VMEM((KBUF, CHUNK, DH), jnp.float32),
            pltpu.VMEM((ZROWS, DH), jnp.float32),
            pltpu.SemaphoreType.DMA,
            pltpu.SemaphoreType.DMA,
            pltpu.SemaphoreType.DMA,
        ],
    )(y2, srcr, dstr)


def kernel(x, edge_index, W, b):
    wt = W.T
    wts = jnp.stack([wt[:, :DH], wt[:, DH:]])
    bs = jnp.stack([b[:DH], b[DH:]])
    y2 = _linear(x, wts, bs)

    pad_len = E_PAD - E
    spread = jnp.arange(pad_len, dtype=jnp.int32)
    src = jnp.concatenate([edge_index[0], spread % N])
    dst = jnp.concatenate([edge_index[1], TRASH + (spread % (N_PAD - N))])
    srcr = src.reshape(NS, NBLK, KBUF, CHUNK)
    dstr = dst.reshape(NS, NBLK, KBUF, CHUNK)

    out2 = _spmm3(y2, srcr, dstr)
    return jnp.concatenate([out2[0], out2[1]], axis=1)


# speedup vs baseline: 1.3068x; 1.3068x over previous
import jax
import jax.numpy as jnp
from jax import lax
from jax.experimental import pallas as pl
from jax.experimental.pallas import tpu as pltpu
from jax.experimental.pallas import tpu_sc as plsc

N = 10000
E = 320000
D = 128
DH = 64
NC = 2
NS = 16
CHUNK = 32
KBUF = 20
NBLK = 32
NCHUNK = NBLK * KBUF
EPT = NCHUNK * CHUNK
E_PAD = EPT * NS
SH = 625
ROWS_PT = 640
N_PAD = ROWS_PT * NS
TRASH = N
ZROWS = 64


def _mm_body(x_ref, wt_ref, b_ref, o_ref):
    xb = x_ref[...]
    for c in range(NC):
        o_ref[c] = (
            jnp.dot(xb, wt_ref[c], preferred_element_type=jnp.float32)
            + b_ref[c][None, :]
        )


def _linear(x, wts, bs):
    bn = 400
    grid = N // bn
    return pl.pallas_call(
        _mm_body,
        grid=(grid,),
        in_specs=[
            pl.BlockSpec((bn, D), lambda i: (i, 0)),
            pl.BlockSpec((NC, D, DH), lambda i: (0, 0, 0)),
            pl.BlockSpec((NC, DH), lambda i: (0, 0)),
        ],
        out_specs=pl.BlockSpec((NC, bn, DH), lambda i: (0, i, 0)),
        out_shape=jax.ShapeDtypeStruct((NC, N, DH), jnp.float32),
    )(x, wts, bs)


def _sc_body(y2, srcr, dstr, out2, tabS, acc, idx_b, rows_v, zero_v,
             sem_i, sem_g, sem_s):
    c = lax.axis_index("c")
    s = lax.axis_index("s")

    def _zfill(r, carry):
        for t in range(DH // 16):
            zero_v[r, pl.ds(t * 16, 16)] = jnp.zeros((16,), jnp.float32)
        return carry

    lax.fori_loop(0, ZROWS, _zfill, 0)

    def clear(tab):
        for z in range(ROWS_PT // ZROWS):
            pltpu.sync_copy(
                zero_v, tab.at[pl.ds(s * ROWS_PT + z * ZROWS, ZROWS)])

    pltpu.sync_copy(y2.at[c, pl.ds(s * SH, SH)], tabS.at[pl.ds(s * SH, SH)])
    clear(acc)
    plsc.subcore_barrier()

    def one_round(tab, ac):
        pltpu.sync_copy(srcr.at[s, 0], idx_b.at[0, 0])
        pltpu.sync_copy(dstr.at[s, 0], idx_b.at[0, 1])
        for k in range(KBUF):
            pltpu.async_copy(tab.at[idx_b.at[0, 0, k]], rows_v.at[k],
                             sem_g)

        def do_block(b, p, fire_next):
            if fire_next:
                pi = pltpu.async_copy(srcr.at[s, b + 1], idx_b.at[1 - p, 0],
                                      sem_i)
                pd = pltpu.async_copy(dstr.at[s, b + 1], idx_b.at[1 - p, 1],
                                      sem_i)
            gathers = []
            scatters = []
            for k in range(KBUF):
                pltpu.make_async_copy(
                    tab.at[idx_b.at[p, 0, k]], rows_v.at[k], sem_g).wait()
                scatters.append(pltpu.async_copy(
                    rows_v.at[k], ac.at[idx_b.at[p, 1, k]], sem_s,
                    add=True))
            if fire_next:
                pi.wait()
                pd.wait()
            for k in range(KBUF):
                scatters[k].wait()
                if fire_next:
                    pltpu.async_copy(
                        tab.at[idx_b.at[1 - p, 0, k]], rows_v.at[k], sem_g)

        def pair(bp, carry):
            do_block(bp * 2, 0, True)
            do_block(bp * 2 + 1, 1, True)
            return carry

        lax.fori_loop(0, NBLK // 2 - 1, pair, 0)
        do_block(NBLK - 2, 0, True)
        do_block(NBLK - 1, 1, False)
        plsc.subcore_barrier()

    one_round(tabS, acc)
    clear(tabS)
    plsc.subcore_barrier()
    one_round(acc, tabS)
    clear(acc)
    plsc.subcore_barrier()
    one_round(tabS, acc)

    pltpu.sync_copy(acc.at[pl.ds(s * SH, SH)],
                    out2.at[c, pl.ds(s * SH, SH)])


def _spmm3(y2, srcr, dstr):
    mesh = plsc.VectorSubcoreMesh(core_axis_name="c", subcore_axis_name="s")
    return pl.kernel(
        _sc_body,
        out_type=jax.ShapeDtypeStruct((NC, N, DH), jnp.float32),
        mesh=mesh,
        compiler_params=pltpu.CompilerParams(use_tc_tiling_on_sc=False),
        scratch_types=[
            pltpu.VMEM_SHARED((N_PAD, DH), jnp.float32),
            pltpu.VMEM_SHARED((N_PAD, DH), jnp.float32),
            pltpu.VMEM((2, 2, KBUF, CHUNK), jnp.int32),
            pltpu.VMEM((KBUF, CHUNK, DH), jnp.float32),
            pltpu.VMEM((ZROWS, DH), jnp.float32),
            pltpu.SemaphoreType.DMA,
            pltpu.SemaphoreType.DMA,
            pltpu.SemaphoreType.DMA,
        ],
    )(y2, srcr, dstr)


def kernel(x, edge_index, W, b):
    wt = W.T
    wts = jnp.stack([wt[:, :DH], wt[:, DH:]])
    bs = jnp.stack([b[:DH], b[DH:]])
    y2 = _linear(x, wts, bs)

    pad_len = E_PAD - E
    spread = jnp.arange(pad_len, dtype=jnp.int32)
    src = jnp.concatenate([edge_index[0], spread % N])
    dst = jnp.concatenate([edge_index[1], TRASH + (spread % (N_PAD - N))])
    srcr = src.reshape(NS, NBLK, KBUF, CHUNK)
    dstr = dst.reshape(NS, NBLK, KBUF, CHUNK)

    out2 = _spmm3(y2, srcr, dstr)
    return jnp.concatenate([out2[0], out2[1]], axis=1)


# speedup vs baseline: 1.3132x; 1.0049x over previous
import jax
import jax.numpy as jnp
from jax import lax
from jax.experimental import pallas as pl
from jax.experimental.pallas import tpu as pltpu
from jax.experimental.pallas import tpu_sc as plsc

N = 10000
E = 320000
D = 128
DH = 64
NC = 2
NS = 16
CHUNK = 64
KBUF = 10
NBLK = 32
NCHUNK = NBLK * KBUF
EPT = NCHUNK * CHUNK
E_PAD = EPT * NS
SH = 625
ROWS_PT = 640
N_PAD = ROWS_PT * NS
TRASH = N
ZROWS = 64


def _mm_body(x_ref, wt_ref, b_ref, o_ref):
    xb = x_ref[...]
    for c in range(NC):
        o_ref[c] = (
            jnp.dot(xb, wt_ref[c], preferred_element_type=jnp.float32)
            + b_ref[c][None, :]
        )


def _linear(x, wts, bs):
    bn = 400
    grid = N // bn
    return pl.pallas_call(
        _mm_body,
        grid=(grid,),
        in_specs=[
            pl.BlockSpec((bn, D), lambda i: (i, 0)),
            pl.BlockSpec((NC, D, DH), lambda i: (0, 0, 0)),
            pl.BlockSpec((NC, DH), lambda i: (0, 0)),
        ],
        out_specs=pl.BlockSpec((NC, bn, DH), lambda i: (0, i, 0)),
        out_shape=jax.ShapeDtypeStruct((NC, N, DH), jnp.float32),
    )(x, wts, bs)


def _sc_body(y2, srcr, dstr, out2, tabS, acc, idx_b, rows_v, zero_v,
             sem_i, sem_g, sem_s):
    c = lax.axis_index("c")
    s = lax.axis_index("s")

    def _zfill(r, carry):
        for t in range(DH // 16):
            zero_v[r, pl.ds(t * 16, 16)] = jnp.zeros((16,), jnp.float32)
        return carry

    lax.fori_loop(0, ZROWS, _zfill, 0)

    def clear(tab):
        for z in range(ROWS_PT // ZROWS):
            pltpu.sync_copy(
                zero_v, tab.at[pl.ds(s * ROWS_PT + z * ZROWS, ZROWS)])

    pltpu.sync_copy(y2.at[c, pl.ds(s * SH, SH)], tabS.at[pl.ds(s * SH, SH)])
    clear(acc)
    plsc.subcore_barrier()

    def one_round(tab, ac):
        pltpu.sync_copy(srcr.at[s, 0], idx_b.at[0, 0])
        pltpu.sync_copy(dstr.at[s, 0], idx_b.at[0, 1])
        for k in range(KBUF):
            pltpu.async_copy(tab.at[idx_b.at[0, 0, k]], rows_v.at[k],
                             sem_g)

        def do_block(b, p, fire_next):
            if fire_next:
                pi = pltpu.async_copy(srcr.at[s, b + 1], idx_b.at[1 - p, 0],
                                      sem_i)
                pd = pltpu.async_copy(dstr.at[s, b + 1], idx_b.at[1 - p, 1],
                                      sem_i)
            gathers = []
            scatters = []
            for k in range(KBUF):
                pltpu.make_async_copy(
                    tab.at[idx_b.at[p, 0, k]], rows_v.at[k], sem_g).wait()
                scatters.append(pltpu.async_copy(
                    rows_v.at[k], ac.at[idx_b.at[p, 1, k]], sem_s,
                    add=True))
            if fire_next:
                pi.wait()
                pd.wait()
            for k in range(KBUF):
                scatters[k].wait()
                if fire_next:
                    pltpu.async_copy(
                        tab.at[idx_b.at[1 - p, 0, k]], rows_v.at[k], sem_g)

        def pair(bp, carry):
            do_block(bp * 2, 0, True)
            do_block(bp * 2 + 1, 1, True)
            return carry

        lax.fori_loop(0, NBLK // 2 - 1, pair, 0)
        do_block(NBLK - 2, 0, True)
        do_block(NBLK - 1, 1, False)
        plsc.subcore_barrier()

    one_round(tabS, acc)
    clear(tabS)
    plsc.subcore_barrier()
    one_round(acc, tabS)
    clear(acc)
    plsc.subcore_barrier()
    one_round(tabS, acc)

    pltpu.sync_copy(acc.at[pl.ds(s * SH, SH)],
                    out2.at[c, pl.ds(s * SH, SH)])


def _spmm3(y2, srcr, dstr):
    mesh = plsc.VectorSubcoreMesh(core_axis_name="c", subcore_axis_name="s")
    return pl.kernel(
        _sc_body,
        out_type=jax.ShapeDtypeStruct((NC, N, DH), jnp.float32),
        mesh=mesh,
        compiler_params=pltpu.CompilerParams(use_tc_tiling_on_sc=False),
        scratch_types=[
            pltpu.VMEM_SHARED((N_PAD, DH), jnp.float32),
            pltpu.VMEM_SHARED((N_PAD, DH), jnp.float32),
            pltpu.VMEM((2, 2, KBUF, CHUNK), jnp.int32),
            pltpu.VMEM((KBUF, CHUNK, DH), jnp.float32),
            pltpu.VMEM((ZROWS, DH), jnp.float32),
            pltpu.SemaphoreType.DMA,
            pltpu.SemaphoreType.DMA,
            pltpu.SemaphoreType.DMA,
        ],
    )(y2, srcr, dstr)


def kernel(x, edge_index, W, b):
    wt = W.T
    wts = jnp.stack([wt[:, :DH], wt[:, DH:]])
    bs = jnp.stack([b[:DH], b[DH:]])
    y2 = _linear(x, wts, bs)

    pad_len = E_PAD - E
    spread = jnp.arange(pad_len, dtype=jnp.int32)
    src = jnp.concatenate([edge_index[0], spread % N])
    dst = jnp.concatenate([edge_index[1], TRASH + (spread % (N_PAD - N))])
    srcr = src.reshape(NS, NBLK, KBUF, CHUNK)
    dstr = dst.reshape(NS, NBLK, KBUF, CHUNK)

    out2 = _spmm3(y2, srcr, dstr)
    return jnp.concatenate([out2[0], out2[1]], axis=1)
